# Initial kernel scaffold; baseline (speedup 1.0000x reference)
import jax
import jax.numpy as jnp
from jax.experimental import pallas as pl


def kernel(topic_ids, W):
    raise NotImplementedError("write your pallas kernel here")


# speedup vs baseline: 1.6841x; 1.6841x over previous
import functools

import jax
import jax.numpy as jnp
from jax import lax
from jax.experimental import pallas as pl
from jax.experimental.pallas import tpu as pltpu
from jax.experimental.pallas import tpu_sc as plsc

NC, NS = 2, 16
NW = NC * NS
CHUNK = 128
BATCH, HIST, DIM = 16384, 50, 64
TOTAL = BATCH * HIST
PER_W = TOTAL // NW
NCHUNK = PER_W // CHUNK

_mesh = plsc.VectorSubcoreMesh(core_axis_name="c", subcore_axis_name="s")


@functools.partial(
    pl.kernel,
    out_type=jax.ShapeDtypeStruct((TOTAL, DIM), jnp.float32),
    mesh=_mesh,
    scratch_types=[
        pltpu.VMEM((NCHUNK, CHUNK), jnp.int32),
        pltpu.VMEM((CHUNK, DIM), jnp.float32),
        pltpu.SemaphoreType.DMA,
    ],
    compiler_params=pltpu.CompilerParams(use_tc_tiling_on_sc=False),
)
def _gather(idx_hbm, table_hbm, out_hbm, idx_v, rows_v, sem):
    wid = lax.axis_index("s") * NC + lax.axis_index("c")
    pltpu.sync_copy(idx_hbm.at[wid], idx_v)
    base = wid * PER_W

    def body(j):
        pltpu.async_copy(table_hbm.at[idx_v.at[j]], rows_v, sem).wait()
        pltpu.sync_copy(rows_v, out_hbm.at[pl.ds(base + j * CHUNK, CHUNK)])

    pl.loop(0, NCHUNK)(body)


def kernel(topic_ids, W):
    idx = topic_ids.reshape(NW, NCHUNK, CHUNK)
    out = _gather(idx, W)
    return out.reshape(BATCH, HIST, DIM), 0


# speedup vs baseline: 1.8746x; 1.1132x over previous
import functools

import jax
import jax.numpy as jnp
from jax import lax
from jax.experimental import pallas as pl
from jax.experimental.pallas import tpu as pltpu
from jax.experimental.pallas import tpu_sc as plsc

NC, NS = 2, 16
NW = NC * NS
CHUNK = 128
K = 4
GROUP = K * CHUNK
BATCH, HIST, DIM = 16384, 50, 64
TOTAL = BATCH * HIST
PER_W = TOTAL // NW
NCHUNK = PER_W // CHUNK
NG = NCHUNK // K

_mesh = plsc.VectorSubcoreMesh(core_axis_name="c", subcore_axis_name="s")


@functools.partial(
    pl.kernel,
    out_type=jax.ShapeDtypeStruct((TOTAL, DIM), jnp.float32),
    mesh=_mesh,
    scratch_types=[
        pltpu.VMEM((NCHUNK, CHUNK), jnp.int32),
        pltpu.VMEM((2, GROUP, DIM), jnp.float32),
        pltpu.SemaphoreType.DMA((2,)),
        pltpu.SemaphoreType.DMA((2,)),
    ],
    compiler_params=pltpu.CompilerParams(use_tc_tiling_on_sc=False),
)
def _gather(idx_hbm, table_hbm, out_hbm, idx_v, rows_v, sem_g, sem_w):
    wid = lax.axis_index("s") * NC + lax.axis_index("c")
    pltpu.sync_copy(idx_hbm.at[wid], idx_v)
    base = wid * PER_W

    def gather_desc(grp, cur, k):
        return pltpu.make_async_copy(
            table_hbm.at[idx_v.at[grp * K + k]],
            rows_v.at[cur, pl.ds(k * CHUNK, CHUNK)],
            sem_g.at[cur],
        )

    def write_desc(grp, cur):
        return pltpu.make_async_copy(
            rows_v.at[cur],
            out_hbm.at[pl.ds(base + grp * GROUP, GROUP)],
            sem_w.at[cur],
        )

    def start_gathers(grp, cur):
        for k in range(K):
            gather_desc(grp, cur, k).start()

    start_gathers(0, 0)

    def body(gp):
        for cur in range(2):
            grp = gp + cur
            other = 1 - cur
            for k in range(K):
                gather_desc(grp, cur, k).wait()

            @pl.when(grp + 1 < NG)
            def _():
                @pl.when(grp >= 1)
                def _():
                    write_desc(grp - 1, other).wait()

                start_gathers(grp + 1, other)

            write_desc(grp, cur).start()

    pl.loop(0, NG, step=2)(body)
    write_desc(NG - 2, 0).wait()
    write_desc(NG - 1, 1).wait()


def kernel(topic_ids, W):
    idx = topic_ids.reshape(NW, NCHUNK, CHUNK)
    out = _gather(idx, W)
    return out.reshape(BATCH, HIST, DIM), 0
